# baseline (speedup 1.0000x reference)
import functools

import jax
import jax.numpy as jnp
from jax import lax
from jax.experimental import pallas as pl
from jax.experimental.pallas import tpu as pltpu
from jax.experimental.pallas import tpu_sc as plsc

_EMBED = 64
_SEQ = 200
_NC = 2
_NS = 16
_NW = _NC * _NS
_CHUNK = 128
_NBUF = 3
_LANE = 16
_BBLK = 64
_BLKROWS = _BBLK * _SEQ
_HALF = _BLKROWS // 2
_NSPLIT = 4
_WHALF = _HALF // 2


def _gather_body(x_ref, tab_ref, out_ref, idx_v, gbuf_v, abuf_v,
                 sem_io, gsem, ssem):
    wid = lax.axis_index("s") * _NC + lax.axis_index("c")
    blk = wid // 2
    wline = (wid % 2) * _WHALF

    xoff = pl.multiple_of(blk * _BLKROWS, _CHUNK)
    pltpu.async_copy(x_ref.at[pl.ds(xoff, _BLKROWS)], idx_v, sem_io).wait()

    npairs = _WHALF // _CHUNK

    def pair_t(p):
        return wline + p * _CHUNK

    def gathers(p, s):
        t = pl.multiple_of(pair_t(p), _CHUNK)
        ia = idx_v.at[pl.ds(t, _CHUNK)]
        ib = idx_v.at[pl.ds(t + _HALF, _CHUNK)]
        ca = pltpu.make_async_copy(tab_ref.at[ia], gbuf_v.at[s, 0], gsem.at[s])
        cb = pltpu.make_async_copy(tab_ref.at[ib], gbuf_v.at[s, 1], gsem.at[s])
        return ca, cb

    def issue_gathers(p, s):
        ca, cb = gathers(p, s)
        ca.start()
        cb.start()

    def wait_gathers(p, s):
        ca, cb = gathers(p, s)
        ca.wait()
        cb.wait()

    def scatter(p, s):
        line0 = pl.multiple_of(blk * _HALF + pair_t(p), _CHUNK)
        dst = out_ref.at[pl.ds(line0, _CHUNK)]
        return pltpu.make_async_copy(abuf_v.at[s], dst, ssem.at[s])

    def repack(s):
        @plsc.parallel_loop(0, _CHUNK, step=2)
        def _(t):
            for u in range(2):
                for k in range(_EMBED // _LANE):
                    sl = pl.ds(k * _LANE, _LANE)
                    abuf_v[s, t + u, pl.ds(k * _LANE, _LANE)] = \
                        gbuf_v[s, 0, t + u, sl]
                    abuf_v[s, t + u, pl.ds(_EMBED + k * _LANE, _LANE)] = \
                        gbuf_v[s, 1, t + u, sl]

    def do_pair(p, s, wait_prev, issue_next):
        wait_gathers(p, s)
        if wait_prev:
            scatter(p - _NBUF, s).wait()
        repack(s)
        scatter(p, s).start()
        if issue_next:
            issue_gathers(p + 2, (s + 2) % _NBUF)

    issue_gathers(0, 0)
    issue_gathers(1, 1)
    do_pair(0, 0, wait_prev=False, issue_next=True)
    do_pair(1, 1, wait_prev=False, issue_next=True)
    do_pair(2, 2, wait_prev=False, issue_next=True)

    def group_body(g, carry):
        for b in range(_NBUF):
            do_pair(g * _NBUF + b, b, wait_prev=True, issue_next=True)
        return carry

    lax.fori_loop(1, (npairs - 4) // _NBUF, group_body, 0)

    do_pair(npairs - 4, (npairs - 4) % _NBUF, wait_prev=True, issue_next=True)
    do_pair(npairs - 3, (npairs - 3) % _NBUF, wait_prev=True, issue_next=True)
    do_pair(npairs - 2, (npairs - 2) % _NBUF, wait_prev=True, issue_next=False)
    do_pair(npairs - 1, (npairs - 1) % _NBUF, wait_prev=True, issue_next=False)
    for p in (npairs - 3, npairs - 2, npairs - 1):
        scatter(p, p % _NBUF).wait()


def _epilogue_body(g_ref, pos_ref, prev_ref, out_ref):
    del prev_ref
    a = g_ref[...]
    lo = a[:, :_EMBED]
    hi = a[:, _EMBED:]
    half = _BBLK // 2
    blk = jnp.concatenate(
        [lo.reshape(half, _SEQ, _EMBED), hi.reshape(half, _SEQ, _EMBED)],
        axis=0)
    out_ref[...] = blk + pos_ref[...][None, :, :]


@jax.jit
def _run(x_flat, pos2d, token_table):
    rows = x_flat.shape[0]
    qrows = rows // _NSPLIT
    nseq = rows // _SEQ
    qblocks = nseq // _BBLK // _NSPLIT
    mesh = plsc.VectorSubcoreMesh(core_axis_name="c", subcore_axis_name="s")

    sck = pl.kernel(
        _gather_body,
        mesh=mesh,
        out_type=jax.ShapeDtypeStruct((qrows // 2, 2 * _EMBED), jnp.float32),
        scratch_types=[
            pltpu.VMEM((_BLKROWS,), jnp.int32),
            pltpu.VMEM((_NBUF, 2, _CHUNK, _EMBED), jnp.float32),
            pltpu.VMEM((_NBUF, _CHUNK, 2 * _EMBED), jnp.float32),
            pltpu.SemaphoreType.DMA,
            pltpu.SemaphoreType.DMA((_NBUF,)),
            pltpu.SemaphoreType.DMA((_NBUF,)),
        ],
        compiler_params=pltpu.CompilerParams(use_tc_tiling_on_sc=False),
    )

    gathered = [sck(lax.slice_in_dim(x_flat, h * qrows, (h + 1) * qrows),
                    token_table)
                for h in range(_NSPLIT)]

    out = None
    for h in range(_NSPLIT):
        operands = [gathered[h], pos2d]
        in_specs = [
            pl.BlockSpec((_HALF, 2 * _EMBED), lambda i: (i, 0)),
            pl.BlockSpec((_SEQ, _EMBED), lambda i: (0, 0)),
        ]
        if out is None:
            prev = jnp.zeros((8,), jnp.float32)
            in_specs.append(pl.BlockSpec(memory_space=pl.ANY))
            aliases = {}
        else:
            prev = out
            in_specs.append(pl.BlockSpec(memory_space=pl.ANY))
            aliases = {2: 0}
        base = h * qblocks
        out = pl.pallas_call(
            _epilogue_body,
            grid=(qblocks,),
            in_specs=in_specs,
            out_specs=pl.BlockSpec(
                (_BBLK, _SEQ, _EMBED),
                functools.partial(lambda bb, i: (bb + i, 0, 0), base)),
            out_shape=jax.ShapeDtypeStruct((nseq, _SEQ, _EMBED), jnp.float32),
            input_output_aliases=aliases,
        )(*operands, prev)
    return out


def kernel(x, token_table, pos_table):
    b, l = x.shape
    x_flat = x.reshape(b * l).astype(jnp.int32)
    pos2d = pos_table[:l]
    return _run(x_flat, pos2d, token_table)


# speedup vs baseline: 1.0803x; 1.0803x over previous
import functools

import jax
import jax.numpy as jnp
from jax import lax
from jax.experimental import pallas as pl
from jax.experimental.pallas import tpu as pltpu
from jax.experimental.pallas import tpu_sc as plsc

_EMBED = 64
_PAD = 128
_SEQ = 200
_NC = 2
_NS = 16
_NW = _NC * _NS
_C0 = 128
_C1 = _SEQ - _C0
_LANE = 16


def _emb_body(seqs_per_worker, x_ref, pos_ref, tab_ref, out_ref,
              idx_v, pos_v, gbuf_v, abuf_v, sem_io, gsem, ssem):
    wid = lax.axis_index("s") * _NC + lax.axis_index("c")
    half = seqs_per_worker // 2 * _SEQ
    wbase = wid * seqs_per_worker

    def stage_idx(h):
        off = pl.multiple_of(wbase * _SEQ + h * half, 8)
        pltpu.async_copy(x_ref.at[pl.ds(off, half)], idx_v, sem_io).wait()

    stage_idx(0)
    pltpu.async_copy(pos_ref, pos_v, sem_io).wait()

    def gathers(si, b):
        loc = lax.rem(si, seqs_per_worker // 2)
        base = pl.multiple_of(loc * _SEQ, 8)
        i0 = idx_v.at[pl.ds(base, _C0)]
        i1 = idx_v.at[pl.ds(base + _C0, _C1)]
        c0 = pltpu.make_async_copy(
            tab_ref.at[i0], gbuf_v.at[b, pl.ds(0, _C0)], gsem.at[b])
        c1 = pltpu.make_async_copy(
            tab_ref.at[i1], gbuf_v.at[b, pl.ds(_C0, _C1)], gsem.at[b])
        return c0, c1

    def scatter(si, b):
        bb = wbase + si
        return pltpu.make_async_copy(abuf_v.at[b], out_ref.at[bb], ssem.at[b])

    def add_pos(b):
        @plsc.parallel_loop(0, _SEQ, step=2)
        def _(r):
            for u in range(2):
                row = r + u
                for k in range(_EMBED // _LANE):
                    sl = pl.ds(k * _LANE, _LANE)
                    abuf_v[b, row, sl] = (
                        gbuf_v[b, row, sl]
                        + pos_v[pl.ds(row * _EMBED + k * _LANE, _LANE)]
                    )

    def issue_gathers(si, b):
        c0, c1 = gathers(si, b)
        c0.start()
        c1.start()

    def wait_gathers(si, b):
        c0, c1 = gathers(si, b)
        c0.wait()
        c1.wait()

    def do_seq(si, b, wait_prev_scatter, issue_next):
        wait_gathers(si, b)
        if issue_next:
            issue_gathers(si + 1, 1 - b)
        if wait_prev_scatter:
            scatter(si - 2, b).wait()
        add_pos(b)
        scatter(si, b).start()

    nsw = seqs_per_worker

    def group_body(g, carry):
        si = g * 2
        do_seq(si, 0, wait_prev_scatter=True, issue_next=True)
        do_seq(si + 1, 1, wait_prev_scatter=True, issue_next=True)
        return carry

    issue_gathers(0, 0)
    do_seq(0, 0, wait_prev_scatter=False, issue_next=True)
    do_seq(1, 1, wait_prev_scatter=False, issue_next=True)
    lax.fori_loop(1, nsw // 4 - 1, group_body, 0)
    do_seq(nsw // 2 - 2, 0, wait_prev_scatter=True, issue_next=True)
    do_seq(nsw // 2 - 1, 1, wait_prev_scatter=True, issue_next=False)

    stage_idx(1)
    issue_gathers(nsw // 2, 0)
    do_seq(nsw // 2, 0, wait_prev_scatter=True, issue_next=True)
    do_seq(nsw // 2 + 1, 1, wait_prev_scatter=True, issue_next=True)
    lax.fori_loop(nsw // 4 + 1, nsw // 2 - 1, group_body, 0)
    do_seq(nsw - 2, 0, wait_prev_scatter=True, issue_next=True)
    do_seq(nsw - 1, 1, wait_prev_scatter=True, issue_next=False)
    scatter(nsw - 2, 0).wait()
    scatter(nsw - 1, 1).wait()


@jax.jit
def _run(x_flat, pos_flat, tab_pad):
    rows = x_flat.shape[0]
    nseq = rows // _SEQ
    seqs_per_worker = nseq // _NW
    mesh = plsc.VectorSubcoreMesh(core_axis_name="c", subcore_axis_name="s")
    body = functools.partial(_emb_body, seqs_per_worker)
    fn = pl.kernel(
        body,
        mesh=mesh,
        out_type=jax.ShapeDtypeStruct((nseq, _SEQ, _EMBED), jnp.float32),
        scratch_types=[
            pltpu.VMEM((seqs_per_worker // 2 * _SEQ,), jnp.int32),
            pltpu.VMEM((_SEQ * _EMBED,), jnp.float32),
            pltpu.VMEM((2, _SEQ, _PAD), jnp.float32),
            pltpu.VMEM((2, _SEQ, _EMBED), jnp.float32),
            pltpu.SemaphoreType.DMA,
            pltpu.SemaphoreType.DMA((2,)),
            pltpu.SemaphoreType.DMA((2,)),
        ],
        compiler_params=pltpu.CompilerParams(use_tc_tiling_on_sc=True),
    )
    return fn(x_flat, pos_flat, tab_pad)


def kernel(x, token_table, pos_table):
    b, l = x.shape
    e = token_table.shape[1]
    x_flat = x.reshape(b * l).astype(jnp.int32)
    pos_flat = pos_table[:l].reshape(-1)
    tab_pad = jnp.pad(token_table, ((0, 0), (0, _PAD - e)))
    return _run(x_flat, pos_flat, tab_pad)


# speedup vs baseline: 1.1047x; 1.0225x over previous
import functools

import jax
import jax.numpy as jnp
from jax import lax
from jax.experimental import pallas as pl
from jax.experimental.pallas import tpu as pltpu
from jax.experimental.pallas import tpu_sc as plsc

_EMBED = 64
_PAD = 128
_SEQ = 200
_NC = 2
_NS = 16
_NW = _NC * _NS
_C0 = 128
_C1 = _SEQ - _C0
_LANE = 16


def _emb_body(seqs_per_worker, x_ref, pos_ref, tab_ref, out_ref,
              idx_v, pos_v, gbuf_v, abuf_v, sem_io, gsem, ssem):
    wid = lax.axis_index("s") * _NC + lax.axis_index("c")
    half = seqs_per_worker // 2 * _SEQ
    wbase = wid * seqs_per_worker

    def stage_idx(h):
        off = pl.multiple_of(wbase * _SEQ + h * half, 8)
        pltpu.async_copy(x_ref.at[pl.ds(off, half)], idx_v, sem_io).wait()

    stage_idx(0)
    pltpu.async_copy(pos_ref, pos_v, sem_io).wait()

    def gathers(si, b):
        loc = lax.rem(si, seqs_per_worker // 2)
        base = pl.multiple_of(loc * _SEQ, 8)
        i0 = idx_v.at[pl.ds(base, _C0)]
        i1 = idx_v.at[pl.ds(base + _C0, _C1)]
        c0 = pltpu.make_async_copy(
            tab_ref.at[i0], gbuf_v.at[b, pl.ds(0, _C0)], gsem.at[b])
        c1 = pltpu.make_async_copy(
            tab_ref.at[i1], gbuf_v.at[b, pl.ds(_C0, _C1)], gsem.at[b])
        return c0, c1

    def scatter(si, b):
        bb = wbase + si
        return pltpu.make_async_copy(abuf_v.at[b], out_ref.at[bb], ssem.at[b])

    def add_pos(b):
        @plsc.parallel_loop(0, _SEQ, step=2)
        def _(r):
            for u in range(2):
                row = r + u
                for k in range(_EMBED // _LANE):
                    sl = pl.ds(k * _LANE, _LANE)
                    abuf_v[b, row, sl] = (
                        gbuf_v[b, row, sl]
                        + pos_v[pl.ds(row * _EMBED + k * _LANE, _LANE)]
                    )

    def issue_gathers(si, b):
        c0, c1 = gathers(si, b)
        c0.start()
        c1.start()

    def wait_gathers(si, b):
        c0, c1 = gathers(si, b)
        c0.wait()
        c1.wait()

    def do_seq(si, b, wait_prev_scatter, issue_next):
        if issue_next:
            issue_gathers(si + 1, 1 - b)
        wait_gathers(si, b)
        if wait_prev_scatter:
            scatter(si - 2, b).wait()
        add_pos(b)
        scatter(si, b).start()

    nsw = seqs_per_worker

    def group_body(g, carry):
        si = g * 2
        do_seq(si, 0, wait_prev_scatter=True, issue_next=True)
        do_seq(si + 1, 1, wait_prev_scatter=True, issue_next=True)
        return carry

    issue_gathers(0, 0)
    do_seq(0, 0, wait_prev_scatter=False, issue_next=True)
    do_seq(1, 1, wait_prev_scatter=False, issue_next=True)
    lax.fori_loop(1, nsw // 4 - 1, group_body, 0)
    do_seq(nsw // 2 - 2, 0, wait_prev_scatter=True, issue_next=True)
    do_seq(nsw // 2 - 1, 1, wait_prev_scatter=True, issue_next=False)

    stage_idx(1)
    issue_gathers(nsw // 2, 0)
    do_seq(nsw // 2, 0, wait_prev_scatter=True, issue_next=True)
    do_seq(nsw // 2 + 1, 1, wait_prev_scatter=True, issue_next=True)
    lax.fori_loop(nsw // 4 + 1, nsw // 2 - 1, group_body, 0)
    do_seq(nsw - 2, 0, wait_prev_scatter=True, issue_next=True)
    do_seq(nsw - 1, 1, wait_prev_scatter=True, issue_next=False)
    scatter(nsw - 2, 0).wait()
    scatter(nsw - 1, 1).wait()


@jax.jit
def _run(x_flat, pos_flat, tab_pad):
    rows = x_flat.shape[0]
    nseq = rows // _SEQ
    seqs_per_worker = nseq // _NW
    mesh = plsc.VectorSubcoreMesh(core_axis_name="c", subcore_axis_name="s")
    body = functools.partial(_emb_body, seqs_per_worker)
    fn = pl.kernel(
        body,
        mesh=mesh,
        out_type=jax.ShapeDtypeStruct((nseq, _SEQ, _EMBED), jnp.float32),
        scratch_types=[
            pltpu.VMEM((seqs_per_worker // 2 * _SEQ,), jnp.int32),
            pltpu.VMEM((_SEQ * _EMBED,), jnp.float32),
            pltpu.VMEM((2, _SEQ, _PAD), jnp.float32),
            pltpu.VMEM((2, _SEQ, _EMBED), jnp.float32),
            pltpu.SemaphoreType.DMA,
            pltpu.SemaphoreType.DMA((2,)),
            pltpu.SemaphoreType.DMA((2,)),
        ],
        compiler_params=pltpu.CompilerParams(use_tc_tiling_on_sc=True),
    )
    return fn(x_flat, pos_flat, tab_pad)


def kernel(x, token_table, pos_table):
    b, l = x.shape
    e = token_table.shape[1]
    x_flat = x.reshape(b * l).astype(jnp.int32)
    pos_flat = pos_table[:l].reshape(-1)
    tab_pad = jnp.pad(token_table, ((0, 0), (0, _PAD - e)))
    return _run(x_flat, pos_flat, tab_pad)
